# Initial kernel scaffold; baseline (speedup 1.0000x reference)
import jax
import jax.numpy as jnp
from jax.experimental import pallas as pl


def kernel(x, weight):
    raise NotImplementedError("write your pallas kernel here")


# speedup vs baseline: 3.2056x; 3.2056x over previous
import functools

import jax
import jax.numpy as jnp
from jax import lax
from jax.experimental import pallas as pl
from jax.experimental.pallas import tpu as pltpu
from jax.experimental.pallas import tpu_sc as plsc

NUM_EMB = 100000
DIM = 64
SCALE = 10.0

NC = 2
NS = 16
NW = NC * NS

LIST = 128
G = 5
CH = G * LIST


@jax.jit
def _embed(idx, weight):
    B = idx.shape[0]
    b_per_w = B // NW
    lists_per_w = b_per_w // LIST
    n_groups = lists_per_w // G

    mesh = plsc.VectorSubcoreMesh(core_axis_name="c", subcore_axis_name="s")

    @functools.partial(
        pl.kernel,
        out_type=jax.ShapeDtypeStruct((B, DIM), jnp.float32),
        mesh=mesh,
        scratch_types=[
            pltpu.VMEM((b_per_w,), jnp.int32),
            pltpu.VMEM((CH, DIM), jnp.float32),
            pltpu.SemaphoreType.DMA,
        ],
        compiler_params=pltpu.CompilerParams(use_tc_tiling_on_sc=False),
    )
    def k(table_hbm, idx_hbm, out_hbm, idx_v, rows_v, sem):
        wid = lax.axis_index("s") * NC + lax.axis_index("c")
        base = wid * b_per_w
        pltpu.sync_copy(idx_hbm.at[pl.ds(base, b_per_w)], idx_v)

        for g in range(n_groups):
            copies = []
            for j in range(G):
                c = pltpu.async_copy(
                    table_hbm.at[idx_v.at[pl.ds((g * G + j) * LIST, LIST)]],
                    rows_v.at[pl.ds(j * LIST, LIST)],
                    sem,
                )
                copies.append(c)
            for c in copies:
                c.wait()

            def scale_row(r, carry):
                for j in range(DIM // 16):
                    sl = pl.ds(j * 16, 16)
                    rows_v[r, sl] = rows_v[r, sl] * SCALE
                return carry

            lax.fori_loop(0, CH, scale_row, 0)

            pltpu.sync_copy(rows_v, out_hbm.at[pl.ds(base + g * CH, CH)])

    return k(weight, idx)


def kernel(x, weight):
    bsz, hist = x.shape
    idx = x.reshape(-1).astype(jnp.int32)
    out = _embed(idx, weight)
    return out.reshape(bsz, hist, DIM)


# speedup vs baseline: 3.6240x; 1.1305x over previous
import functools

import jax
import jax.numpy as jnp
from jax import lax
from jax.experimental import pallas as pl
from jax.experimental.pallas import tpu as pltpu
from jax.experimental.pallas import tpu_sc as plsc

NUM_EMB = 100000
DIM = 64
SCALE = 10.0

NC = 2
NS = 16
NW = NC * NS

LIST = 128
G = 5
CH = G * LIST


@jax.jit
def _embed(idx, weight):
    B = idx.shape[0]
    b_per_w = B // NW
    lists_per_w = b_per_w // LIST
    n_groups = lists_per_w // G

    mesh = plsc.VectorSubcoreMesh(core_axis_name="c", subcore_axis_name="s")

    @functools.partial(
        pl.kernel,
        out_type=jax.ShapeDtypeStruct((B, DIM), jnp.float32),
        mesh=mesh,
        scratch_types=[
            pltpu.VMEM((b_per_w,), jnp.int32),
            pltpu.VMEM((CH, DIM), jnp.float32),
            pltpu.VMEM((CH, DIM), jnp.float32),
            pltpu.SemaphoreType.DMA,
            pltpu.SemaphoreType.DMA,
            pltpu.SemaphoreType.DMA,
            pltpu.SemaphoreType.DMA,
        ],
        compiler_params=pltpu.CompilerParams(use_tc_tiling_on_sc=False),
    )
    def k(table_hbm, idx_hbm, out_hbm, idx_v, rows0, rows1, in0, in1, o0, o1):
        bufs = (rows0, rows1)
        ins = (in0, in1)
        outs = (o0, o1)
        wid = lax.axis_index("s") * NC + lax.axis_index("c")
        base = wid * b_per_w
        pltpu.sync_copy(idx_hbm.at[pl.ds(base, b_per_w)], idx_v)

        def fire_gather(g, buf, sem):
            return [
                pltpu.async_copy(
                    table_hbm.at[idx_v.at[pl.ds((g * G + j) * LIST, LIST)]],
                    buf.at[pl.ds(j * LIST, LIST)],
                    sem,
                )
                for j in range(G)
            ]

        pend_in = {0: fire_gather(0, bufs[0], ins[0])}
        pend_out = {}
        for g in range(n_groups):
            b = g % 2
            if g + 1 < n_groups:
                nb = (g + 1) % 2
                if g - 1 >= 0:
                    for c in pend_out[g - 1]:
                        c.wait()
                pend_in[g + 1] = fire_gather(g + 1, bufs[nb], ins[nb])
            for c in pend_in[g]:
                c.wait()

            buf = bufs[b]

            @plsc.parallel_loop(0, CH, unroll=4)
            def scale_row(r):
                for j in range(DIM // 16):
                    sl = pl.ds(j * 16, 16)
                    buf[r, sl] = buf[r, sl] * SCALE

            pend_out[g] = [
                pltpu.async_copy(
                    buf, out_hbm.at[pl.ds(base + g * CH, CH)], outs[b]
                )
            ]
        for g in (n_groups - 2, n_groups - 1):
            for c in pend_out[g]:
                c.wait()

    return k(weight, idx)


def kernel(x, weight):
    bsz, hist = x.shape
    idx = x.reshape(-1).astype(jnp.int32)
    out = _embed(idx, weight)
    return out.reshape(bsz, hist, DIM)
